# Initial kernel scaffold; baseline (speedup 1.0000x reference)
import jax
import jax.numpy as jnp
from jax.experimental import pallas as pl


def kernel(x, importance, W_proj, b_proj, neuron_emb):
    raise NotImplementedError("write your pallas kernel here")


# speedup vs baseline: 9.3620x; 9.3620x over previous
import jax
import jax.numpy as jnp
from jax.experimental import pallas as pl
from jax.experimental.pallas import tpu as pltpu

_GROUP = 64
_N_USED = 256
_GROUPS = ((0, 8, 1), (64, 8, 1), (128, 4, 2), (192, 6, 1))
_TB = 512

_NEG = -3.0e38


def _route_kernel(x_ref, w_ref, b_ref, emb_ref,
                  fr_ref, fv_ref, rq_ref, rk_ref, val_ref,
                  embn_s):
    @pl.when(pl.program_id(0) == 0)
    def _prep():
        emb = emb_ref[...]
        nrm = jnp.sqrt(jnp.sum(emb * emb, axis=1, keepdims=True))
        embn_s[...] = emb / jnp.maximum(nrm, 1e-12)

    h = jnp.dot(x_ref[...], w_ref[...],
                preferred_element_type=jnp.float32) + b_ref[...]
    logits = jax.lax.dot_general(
        h, embn_s[...], (((1,), (1,)), ((), ())),
        preferred_element_type=jnp.float32)

    outs = (fr_ref, fv_ref, rq_ref, rk_ref, val_ref)
    oi = 0
    for off, k, nouts in _GROUPS:
        lg = logits[:, off:off + _GROUP]
        m = jnp.max(lg, axis=1, keepdims=True)
        e = jnp.exp(lg - m)
        z = jnp.sum(e, axis=1, keepdims=True)
        p = e / z
        work = lg
        thr = m
        for _ in range(k):
            thr = jnp.max(work, axis=1, keepdims=True)
            work = jnp.where(work >= thr, _NEG, work)
        sparse = jnp.where(lg >= thr, p, 0.0)
        out = sparse / (jnp.sum(sparse, axis=1, keepdims=True) + 1e-8)
        for _ in range(nouts):
            outs[oi][...] = out
            oi += 1


def kernel(x, importance, W_proj, b_proj, neuron_emb):
    del importance
    b, s, d = x.shape
    t = b * s
    x2 = x.reshape(t, d)
    grid = (t // _TB,)
    out_sds = [jax.ShapeDtypeStruct((t, _GROUP), jnp.float32)] * 5
    outs = pl.pallas_call(
        _route_kernel,
        grid=grid,
        in_specs=[
            pl.BlockSpec((_TB, d), lambda i: (i, 0)),
            pl.BlockSpec((d, _GROUP), lambda i: (0, 0)),
            pl.BlockSpec((1, _GROUP), lambda i: (0, 0)),
            pl.BlockSpec((_N_USED, _GROUP), lambda i: (0, 0)),
        ],
        out_specs=[pl.BlockSpec((_TB, _GROUP), lambda i: (i, 0))] * 5,
        out_shape=out_sds,
        scratch_shapes=[
            pltpu.VMEM((_N_USED, _GROUP), jnp.float32),
        ],
        compiler_params=pltpu.CompilerParams(
            dimension_semantics=("arbitrary",),
        ),
    )(x2, W_proj, b_proj.reshape(1, _GROUP), neuron_emb[:_N_USED])
    return tuple(o.reshape(b, s, _GROUP) for o in outs)


# speedup vs baseline: 17.7781x; 1.8990x over previous
import jax
import jax.numpy as jnp
from jax.experimental import pallas as pl
from jax.experimental.pallas import tpu as pltpu

_GROUP = 64
_N_USED = 256
_TB = 512

_NEG = -3.0e38


def _route_kernel(x_ref, w_ref, b_ref, emb_ref,
                  fr_ref, fv_ref, rq_ref, rk_ref, val_ref,
                  embn_s):
    tb = x_ref.shape[0]

    @pl.when(pl.program_id(0) == 0)
    def _prep():
        emb = emb_ref[...]
        nrm = jnp.sqrt(jnp.sum(emb * emb, axis=1, keepdims=True))
        embn_s[...] = emb / jnp.maximum(nrm, 1e-12)

    h = jnp.dot(x_ref[...], w_ref[...],
                preferred_element_type=jnp.float32) + b_ref[...]
    lgt = jax.lax.dot_general(
        embn_s[...], h, (((1,), (1,)), ((), ())),
        preferred_element_type=jnp.float32)
    lg4 = lgt.reshape(4, _GROUP, tb)

    m = jnp.max(lg4, axis=1, keepdims=True)
    e = jnp.exp(lg4 - m)
    z = jnp.sum(e, axis=1, keepdims=True)

    cur = m
    work = jnp.where(lg4 >= m, _NEG, lg4)
    for _ in range(2, 5):
        cur = jnp.max(work, axis=1, keepdims=True)
        work = jnp.where(work >= cur, _NEG, work)
    thr_rel = cur[0:1]
    work = work[1:]
    for _ in range(5, 7):
        cur = jnp.max(work, axis=1, keepdims=True)
        work = jnp.where(work >= cur, _NEG, work)
    thr_val = cur[0:1]
    work = work[1:]
    cur = jnp.max(work, axis=1, keepdims=True)
    work = jnp.where(work >= cur, _NEG, work)
    cur = jnp.max(work, axis=1, keepdims=True)
    thr = jnp.concatenate([thr_rel, thr_val, cur], axis=0)

    sparse = jnp.where(lg4 >= thr, e, 0.0)
    s = jnp.sum(sparse, axis=1, keepdims=True)
    scale = 1.0 / (s + 1e-8 * z)
    outt = (sparse * scale).reshape(_N_USED, tb)
    outp = jax.lax.transpose(outt, (1, 0))

    rq_ref[...] = outp[:, 0:64]
    rk_ref[...] = outp[:, 0:64]
    val_ref[...] = outp[:, 64:128]
    fr_ref[...] = outp[:, 128:192]
    fv_ref[...] = outp[:, 192:256]


def kernel(x, importance, W_proj, b_proj, neuron_emb):
    del importance
    b, s, d = x.shape
    t = b * s
    x2 = x.reshape(t, d)
    emb_r = jnp.concatenate([
        neuron_emb[128:192], neuron_emb[192:256],
        neuron_emb[0:64], neuron_emb[64:128]], axis=0)
    grid = (t // _TB,)
    out_sds = [jax.ShapeDtypeStruct((t, _GROUP), jnp.float32)] * 5
    outs = pl.pallas_call(
        _route_kernel,
        grid=grid,
        in_specs=[
            pl.BlockSpec((_TB, d), lambda i: (i, 0)),
            pl.BlockSpec((d, _GROUP), lambda i: (0, 0)),
            pl.BlockSpec((1, _GROUP), lambda i: (0, 0)),
            pl.BlockSpec((_N_USED, _GROUP), lambda i: (0, 0)),
        ],
        out_specs=[pl.BlockSpec((_TB, _GROUP), lambda i: (i, 0))] * 5,
        out_shape=out_sds,
        scratch_shapes=[
            pltpu.VMEM((_N_USED, _GROUP), jnp.float32),
        ],
        compiler_params=pltpu.CompilerParams(
            dimension_semantics=("arbitrary",),
        ),
    )(x2, W_proj, b_proj.reshape(1, _GROUP), emb_r)
    return tuple(o.reshape(b, s, _GROUP) for o in outs)


# speedup vs baseline: 19.7940x; 1.1134x over previous
import jax
import jax.numpy as jnp
from jax.experimental import pallas as pl
from jax.experimental.pallas import tpu as pltpu

_GROUP = 64
_N_USED = 256
_TB = 1024

_NEG = -3.0e38


def _route_kernel(x_ref, w_ref, b_ref, emb_ref,
                  fr_ref, fv_ref, rq_ref, rk_ref, val_ref,
                  embn_s):
    tb = x_ref.shape[0]

    @pl.when(pl.program_id(0) == 0)
    def _prep():
        emb = emb_ref[...]
        nrm = jnp.sqrt(jnp.sum(emb * emb, axis=1, keepdims=True))
        embn_s[...] = emb / jnp.maximum(nrm, 1e-12)

    h = jnp.dot(x_ref[...], w_ref[...],
                preferred_element_type=jnp.float32) + b_ref[...]
    lgt = jax.lax.dot_general(
        embn_s[...], h, (((1,), (1,)), ((), ())),
        preferred_element_type=jnp.float32)
    lg4 = lgt.reshape(4, _GROUP, tb)

    m = jnp.max(lg4, axis=1, keepdims=True)
    e = jnp.exp(lg4 - m)
    z = jnp.sum(e, axis=1, keepdims=True)

    cur = m
    work = jnp.where(lg4 >= m, _NEG, lg4)
    for _ in range(2, 5):
        cur = jnp.max(work, axis=1, keepdims=True)
        work = jnp.where(work >= cur, _NEG, work)
    thr_rel = cur[0:1]
    work = work[1:]
    for _ in range(5, 7):
        cur = jnp.max(work, axis=1, keepdims=True)
        work = jnp.where(work >= cur, _NEG, work)
    thr_val = cur[0:1]
    work = work[1:]
    cur = jnp.max(work, axis=1, keepdims=True)
    work = jnp.where(work >= cur, _NEG, work)
    cur = jnp.max(work, axis=1, keepdims=True)
    thr = jnp.concatenate([thr_rel, thr_val, cur], axis=0)

    sparse = jnp.where(lg4 >= thr, e, 0.0)
    s = jnp.sum(sparse, axis=1, keepdims=True)
    scale = 1.0 / (s + 1e-8 * z)
    outt = (sparse * scale).reshape(_N_USED, tb)
    outp = jax.lax.transpose(outt, (1, 0))

    rq_ref[...] = outp[:, 0:64]
    rk_ref[...] = outp[:, 0:64]
    val_ref[...] = outp[:, 64:128]
    fr_ref[...] = outp[:, 128:192]
    fv_ref[...] = outp[:, 192:256]


def kernel(x, importance, W_proj, b_proj, neuron_emb):
    del importance
    b, s, d = x.shape
    t = b * s
    x2 = x.reshape(t, d)
    emb_r = jnp.concatenate([
        neuron_emb[128:192], neuron_emb[192:256],
        neuron_emb[0:64], neuron_emb[64:128]], axis=0)
    grid = (t // _TB,)
    out_sds = [jax.ShapeDtypeStruct((t, _GROUP), jnp.float32)] * 5
    outs = pl.pallas_call(
        _route_kernel,
        grid=grid,
        in_specs=[
            pl.BlockSpec((_TB, d), lambda i: (i, 0)),
            pl.BlockSpec((d, _GROUP), lambda i: (0, 0)),
            pl.BlockSpec((1, _GROUP), lambda i: (0, 0)),
            pl.BlockSpec((_N_USED, _GROUP), lambda i: (0, 0)),
        ],
        out_specs=[pl.BlockSpec((_TB, _GROUP), lambda i: (i, 0))] * 5,
        out_shape=out_sds,
        scratch_shapes=[
            pltpu.VMEM((_N_USED, _GROUP), jnp.float32),
        ],
        compiler_params=pltpu.CompilerParams(
            dimension_semantics=("arbitrary",),
        ),
    )(x2, W_proj, b_proj.reshape(1, _GROUP), emb_r)
    return tuple(o.reshape(b, s, _GROUP) for o in outs)


# speedup vs baseline: 20.0869x; 1.0148x over previous
import jax
import jax.numpy as jnp
from jax.experimental import pallas as pl
from jax.experimental.pallas import tpu as pltpu

_GROUP = 64
_N_USED = 256
_TB = 2048

_NEG = -3.0e38


def _route_kernel(x_ref, w_ref, b_ref, emb_ref,
                  fr_ref, fv_ref, rq_ref, rk_ref, val_ref,
                  embn_s):
    tb = x_ref.shape[0]

    @pl.when(pl.program_id(0) == 0)
    def _prep():
        emb = emb_ref[...]
        nrm = jnp.sqrt(jnp.sum(emb * emb, axis=1, keepdims=True))
        embn_s[...] = emb / jnp.maximum(nrm, 1e-12)

    h = jnp.dot(x_ref[...], w_ref[...],
                preferred_element_type=jnp.float32) + b_ref[...]
    lgt = jax.lax.dot_general(
        embn_s[...], h, (((1,), (1,)), ((), ())),
        preferred_element_type=jnp.float32)
    lg4 = lgt.reshape(4, _GROUP, tb)

    m = jnp.max(lg4, axis=1, keepdims=True)
    e = jnp.exp(lg4 - m)
    z = jnp.sum(e, axis=1, keepdims=True)

    cur = m
    work = jnp.where(lg4 >= m, _NEG, lg4)
    for _ in range(2, 5):
        cur = jnp.max(work, axis=1, keepdims=True)
        work = jnp.where(work >= cur, _NEG, work)
    thr_rel = cur[0:1]
    work = work[1:]
    for _ in range(5, 7):
        cur = jnp.max(work, axis=1, keepdims=True)
        work = jnp.where(work >= cur, _NEG, work)
    thr_val = cur[0:1]
    work = work[1:]
    cur = jnp.max(work, axis=1, keepdims=True)
    work = jnp.where(work >= cur, _NEG, work)
    cur = jnp.max(work, axis=1, keepdims=True)
    thr = jnp.concatenate([thr_rel, thr_val, cur], axis=0)

    sparse = jnp.where(lg4 >= thr, e, 0.0)
    s = jnp.sum(sparse, axis=1, keepdims=True)
    scale = 1.0 / (s + 1e-8 * z)
    outt = (sparse * scale).reshape(_N_USED, tb)
    outp = jax.lax.transpose(outt, (1, 0))

    rq_ref[...] = outp[:, 0:64]
    rk_ref[...] = outp[:, 0:64]
    val_ref[...] = outp[:, 64:128]
    fr_ref[...] = outp[:, 128:192]
    fv_ref[...] = outp[:, 192:256]


def kernel(x, importance, W_proj, b_proj, neuron_emb):
    del importance
    b, s, d = x.shape
    t = b * s
    x2 = x.reshape(t, d)
    emb_r = jnp.concatenate([
        neuron_emb[128:192], neuron_emb[192:256],
        neuron_emb[0:64], neuron_emb[64:128]], axis=0)
    grid = (t // _TB,)
    out_sds = [jax.ShapeDtypeStruct((t, _GROUP), jnp.float32)] * 5
    outs = pl.pallas_call(
        _route_kernel,
        grid=grid,
        in_specs=[
            pl.BlockSpec((_TB, d), lambda i: (i, 0)),
            pl.BlockSpec((d, _GROUP), lambda i: (0, 0)),
            pl.BlockSpec((1, _GROUP), lambda i: (0, 0)),
            pl.BlockSpec((_N_USED, _GROUP), lambda i: (0, 0)),
        ],
        out_specs=[pl.BlockSpec((_TB, _GROUP), lambda i: (i, 0))] * 5,
        out_shape=out_sds,
        scratch_shapes=[
            pltpu.VMEM((_N_USED, _GROUP), jnp.float32),
        ],
        compiler_params=pltpu.CompilerParams(
            dimension_semantics=("arbitrary",),
        ),
    )(x2, W_proj, b_proj.reshape(1, _GROUP), emb_r)
    return tuple(o.reshape(b, s, _GROUP) for o in outs)


# speedup vs baseline: 21.3593x; 1.0633x over previous
import jax
import jax.numpy as jnp
from jax.experimental import pallas as pl
from jax.experimental.pallas import tpu as pltpu

_GROUP = 64
_N_USED = 256
_TB = 1024

_NEG = -3.0e38


def _route_kernel(x_ref, w_ref, b_ref, emb_ref,
                  fr_ref, fv_ref, rq_ref, val_ref,
                  embn_s):
    tb = x_ref.shape[0]

    @pl.when(pl.program_id(0) == 0)
    def _prep():
        emb = emb_ref[...]
        nrm = jnp.sqrt(jnp.sum(emb * emb, axis=1, keepdims=True))
        embn_s[...] = emb / jnp.maximum(nrm, 1e-12)

    h = jnp.dot(x_ref[...], w_ref[...],
                preferred_element_type=jnp.float32) + b_ref[...]
    lgt = jax.lax.dot_general(
        embn_s[...], h, (((1,), (1,)), ((), ())),
        preferred_element_type=jnp.float32)
    lg4 = lgt.reshape(4, _GROUP, tb)

    m = jnp.max(lg4, axis=1, keepdims=True)
    e = jnp.exp(lg4 - m)
    z = jnp.sum(e, axis=1, keepdims=True)

    cur = m
    work = jnp.where(lg4 >= m, _NEG, lg4)
    for _ in range(2, 5):
        cur = jnp.max(work, axis=1, keepdims=True)
        work = jnp.where(work >= cur, _NEG, work)
    thr_rel = cur[0:1]
    work = work[1:]
    for _ in range(5, 7):
        cur = jnp.max(work, axis=1, keepdims=True)
        work = jnp.where(work >= cur, _NEG, work)
    thr_val = cur[0:1]
    work = work[1:]
    cur = jnp.max(work, axis=1, keepdims=True)
    work = jnp.where(work >= cur, _NEG, work)
    cur = jnp.max(work, axis=1, keepdims=True)
    thr = jnp.concatenate([thr_rel, thr_val, cur], axis=0)

    sparse = jnp.where(lg4 >= thr, e, 0.0)
    s = jnp.sum(sparse, axis=1, keepdims=True)
    scale = 1.0 / (s + 1e-8 * z)
    outt = (sparse * scale).reshape(_N_USED, tb)
    outp = jax.lax.transpose(outt, (1, 0))

    rq_ref[...] = outp[:, 0:64]
    val_ref[...] = outp[:, 64:128]
    fr_ref[...] = outp[:, 128:192]
    fv_ref[...] = outp[:, 192:256]


def kernel(x, importance, W_proj, b_proj, neuron_emb):
    del importance
    b, s, d = x.shape
    t = b * s
    x2 = x.reshape(t, d)
    emb_r = jnp.concatenate([
        neuron_emb[128:192], neuron_emb[192:256],
        neuron_emb[0:64], neuron_emb[64:128]], axis=0)
    grid = (t // _TB,)
    out_sds = [jax.ShapeDtypeStruct((t, _GROUP), jnp.float32)] * 4
    outs = pl.pallas_call(
        _route_kernel,
        grid=grid,
        in_specs=[
            pl.BlockSpec((_TB, d), lambda i: (i, 0)),
            pl.BlockSpec((d, _GROUP), lambda i: (0, 0)),
            pl.BlockSpec((1, _GROUP), lambda i: (0, 0)),
            pl.BlockSpec((_N_USED, _GROUP), lambda i: (0, 0)),
        ],
        out_specs=[pl.BlockSpec((_TB, _GROUP), lambda i: (i, 0))] * 4,
        out_shape=out_sds,
        scratch_shapes=[
            pltpu.VMEM((_N_USED, _GROUP), jnp.float32),
        ],
        compiler_params=pltpu.CompilerParams(
            dimension_semantics=("arbitrary",),
        ),
    )(x2, W_proj, b_proj.reshape(1, _GROUP), emb_r)
    fr_o, fv_o, rq_o, val_o = (o.reshape(b, s, _GROUP) for o in outs)
    return (fr_o, fv_o, rq_o, rq_o, val_o)


# speedup vs baseline: 21.6790x; 1.0150x over previous
import jax
import jax.numpy as jnp
from jax.experimental import pallas as pl
from jax.experimental.pallas import tpu as pltpu

_GROUP = 64
_N_USED = 256
_TB = 1920

_NEG = -3.0e38


def _route_kernel(x_ref, w_ref, b_ref, emb_ref,
                  fr_ref, fv_ref, rq_ref, val_ref,
                  embn_s):
    tb = x_ref.shape[0]

    @pl.when(pl.program_id(0) == 0)
    def _prep():
        emb = emb_ref[...]
        nrm = jnp.sqrt(jnp.sum(emb * emb, axis=1, keepdims=True))
        embn_s[...] = emb / jnp.maximum(nrm, 1e-12)

    h = jnp.dot(x_ref[...], w_ref[...],
                preferred_element_type=jnp.float32) + b_ref[...]
    lgt = jax.lax.dot_general(
        embn_s[...], h, (((1,), (1,)), ((), ())),
        preferred_element_type=jnp.float32)
    lg4 = lgt.reshape(4, _GROUP, tb)

    m = jnp.max(lg4, axis=1, keepdims=True)
    e = jnp.exp(lg4 - m)
    z = jnp.sum(e, axis=1, keepdims=True)

    cur = m
    work = jnp.where(lg4 >= m, _NEG, lg4)
    for _ in range(2, 5):
        cur = jnp.max(work, axis=1, keepdims=True)
        work = jnp.where(work >= cur, _NEG, work)
    thr_rel = cur[0:1]
    work = work[1:]
    for _ in range(5, 7):
        cur = jnp.max(work, axis=1, keepdims=True)
        work = jnp.where(work >= cur, _NEG, work)
    thr_val = cur[0:1]
    work = work[1:]
    cur = jnp.max(work, axis=1, keepdims=True)
    work = jnp.where(work >= cur, _NEG, work)
    cur = jnp.max(work, axis=1, keepdims=True)
    thr = jnp.concatenate([thr_rel, thr_val, cur], axis=0)

    sparse = jnp.where(lg4 >= thr, e, 0.0)
    s = jnp.sum(sparse, axis=1, keepdims=True)
    scale = 1.0 / (s + 1e-8 * z)
    outt = (sparse * scale).reshape(_N_USED, tb)
    outp = jax.lax.transpose(outt, (1, 0))

    rq_ref[...] = outp[:, 0:64]
    val_ref[...] = outp[:, 64:128]
    fr_ref[...] = outp[:, 128:192]
    fv_ref[...] = outp[:, 192:256]


def kernel(x, importance, W_proj, b_proj, neuron_emb):
    del importance
    b, s, d = x.shape
    t = b * s
    x2 = x.reshape(t, d)
    emb_r = jnp.concatenate([
        neuron_emb[128:192], neuron_emb[192:256],
        neuron_emb[0:64], neuron_emb[64:128]], axis=0)
    grid = (pl.cdiv(t, _TB),)
    out_sds = [jax.ShapeDtypeStruct((t, _GROUP), jnp.float32)] * 4
    outs = pl.pallas_call(
        _route_kernel,
        grid=grid,
        in_specs=[
            pl.BlockSpec((_TB, d), lambda i: (i, 0)),
            pl.BlockSpec((d, _GROUP), lambda i: (0, 0)),
            pl.BlockSpec((1, _GROUP), lambda i: (0, 0)),
            pl.BlockSpec((_N_USED, _GROUP), lambda i: (0, 0)),
        ],
        out_specs=[pl.BlockSpec((_TB, _GROUP), lambda i: (i, 0))] * 4,
        out_shape=out_sds,
        scratch_shapes=[
            pltpu.VMEM((_N_USED, _GROUP), jnp.float32),
        ],
        compiler_params=pltpu.CompilerParams(
            dimension_semantics=("arbitrary",),
        ),
    )(x2, W_proj, b_proj.reshape(1, _GROUP), emb_r)
    fr_o, fv_o, rq_o, val_o = (o.reshape(b, s, _GROUP) for o in outs)
    return (fr_o, fv_o, rq_o, rq_o, val_o)


# speedup vs baseline: 21.8107x; 1.0061x over previous
import jax
import jax.numpy as jnp
from jax.experimental import pallas as pl
from jax.experimental.pallas import tpu as pltpu

_GROUP = 64
_N_USED = 256
_TB = 2048

_NEG = -3.0e38


def _route_kernel(x_ref, w_ref, b_ref, emb_ref,
                  fr_ref, fv_ref, rq_ref, val_ref,
                  embn_s):
    tb = x_ref.shape[0]

    @pl.when(pl.program_id(0) == 0)
    def _prep():
        emb = emb_ref[...]
        nrm = jnp.sqrt(jnp.sum(emb * emb, axis=1, keepdims=True))
        embn_s[...] = emb / jnp.maximum(nrm, 1e-12)

    h = jnp.dot(x_ref[...], w_ref[...],
                preferred_element_type=jnp.float32) + b_ref[...]
    lgt = jax.lax.dot_general(
        embn_s[...], h, (((1,), (1,)), ((), ())),
        preferred_element_type=jnp.float32)
    lg4 = lgt.reshape(4, _GROUP, tb)

    m = jnp.max(lg4, axis=1, keepdims=True)
    e = jnp.exp(lg4 - m)
    z = jnp.sum(e, axis=1, keepdims=True)

    cur = m
    work = jnp.where(lg4 >= m, _NEG, lg4)
    for _ in range(2, 5):
        cur = jnp.max(work, axis=1, keepdims=True)
        work = jnp.where(work >= cur, _NEG, work)
    thr_rel = cur[0:1]
    work = work[1:]
    for _ in range(5, 7):
        cur = jnp.max(work, axis=1, keepdims=True)
        work = jnp.where(work >= cur, _NEG, work)
    thr_val = cur[0:1]
    work = work[1:]
    cur = jnp.max(work, axis=1, keepdims=True)
    work = jnp.where(work >= cur, _NEG, work)
    cur = jnp.max(work, axis=1, keepdims=True)
    thr = jnp.concatenate([thr_rel, thr_val, cur], axis=0)

    sparse = jnp.where(lg4 >= thr, e, 0.0)
    s = jnp.sum(sparse, axis=1, keepdims=True)
    scale = 1.0 / (s + 1e-8 * z)
    outt = (sparse * scale).reshape(_N_USED, tb)
    outp = jax.lax.transpose(outt, (1, 0))

    rq_ref[...] = outp[:, 0:64]
    val_ref[...] = outp[:, 64:128]
    fr_ref[...] = outp[:, 128:192]
    fv_ref[...] = outp[:, 192:256]


def kernel(x, importance, W_proj, b_proj, neuron_emb):
    del importance
    b, s, d = x.shape
    t = b * s
    x2 = x.reshape(t, d)
    emb_r = jnp.concatenate([
        neuron_emb[128:192], neuron_emb[192:256],
        neuron_emb[0:64], neuron_emb[64:128]], axis=0)
    grid = (t // _TB,)
    out_sds = [jax.ShapeDtypeStruct((t, _GROUP), jnp.float32)] * 4
    outs = pl.pallas_call(
        _route_kernel,
        grid=grid,
        in_specs=[
            pl.BlockSpec((_TB, d), lambda i: (i, 0)),
            pl.BlockSpec((d, _GROUP), lambda i: (0, 0)),
            pl.BlockSpec((1, _GROUP), lambda i: (0, 0)),
            pl.BlockSpec((_N_USED, _GROUP), lambda i: (0, 0)),
        ],
        out_specs=[pl.BlockSpec((_TB, _GROUP), lambda i: (i, 0))] * 4,
        out_shape=out_sds,
        scratch_shapes=[
            pltpu.VMEM((_N_USED, _GROUP), jnp.float32),
        ],
        compiler_params=pltpu.CompilerParams(
            dimension_semantics=("arbitrary",),
        ),
    )(x2, W_proj, b_proj.reshape(1, _GROUP), emb_r)
    fr_o, fv_o, rq_o, val_o = (o.reshape(b, s, _GROUP) for o in outs)
    return (fr_o, fv_o, rq_o, rq_o, val_o)
